# baseline (speedup 1.0000x reference)
import functools

import jax
import jax.numpy as jnp
from jax import lax
from jax.experimental import pallas as pl
from jax.experimental.pallas import tpu as pltpu
from jax.experimental.pallas import tpu_sc as plsc

NODES = 50000
NP = 50048
NUSERS = 25000
D = 64
DH = 32
E = 800000
CH = 128
CHUNKS = 6336
EPAD = CHUNKS * CH
TILES = 16
CPT = CHUNKS // TILES
K = 3
BLOCKS = CPT // K
HBLOCKS = BLOCKS // 2
ROWS_PT = NP // TILES
WB = 136
NWB = ROWS_PT // WB

_mesh = plsc.VectorSubcoreMesh(core_axis_name="c", subcore_axis_name="s")


@functools.partial(
    pl.kernel,
    mesh=_mesh,
    compiler_params=pltpu.CompilerParams(use_tc_tiling_on_sc=False),
    out_type=(
        jax.ShapeDtypeStruct((2 * NP, DH), jnp.float32),
        jax.ShapeDtypeStruct((2 * NP, DH), jnp.float32),
    ),
    scratch_types=[
        pltpu.VMEM((2, K, CH), jnp.int32),
        pltpu.VMEM((2, K, CH), jnp.int32),
        pltpu.VMEM((2, K, CH, DH), jnp.float32),
        pltpu.VMEM((WB, DH), jnp.float32),
        pltpu.VMEM_SHARED((NP, DH), jnp.float32),
        pltpu.SemaphoreType.DMA,
        pltpu.SemaphoreType.DMA,
        pltpu.SemaphoreType.DMA,
        pltpu.SemaphoreType.DMA,
        pltpu.SemaphoreType.DMA,
    ],
)
def _propagate(tab0, rows_hbm, cols_hbm, zeros_hbm, x1_out, x2_out,
               rows_v, cols_v, gbuf, zbuf, acc,
               gsem0, gsem1, ssem0, ssem1, zsem):
    c = lax.axis_index("c")
    s = lax.axis_index("s")
    coff = c * NP
    gsem = (gsem0, gsem1)
    ssem = (ssem0, ssem1)

    pltpu.sync_copy(zeros_hbm, zbuf)

    def zero_acc():
        cps = []
        for k in range(NWB):
            cps.append(pltpu.async_copy(
                zbuf, acc.at[pl.ds(s * ROWS_PT + k * WB, WB)], zsem))
        for cp in cps:
            cp.wait()

    def layer(src_tab, dst_tab):
        zero_acc()
        plsc.subcore_barrier()

        base = s * CPT

        def load_idx(b, g):
            pltpu.sync_copy(rows_hbm.at[pl.ds(base + b * K, K)],
                            rows_v.at[g])
            pltpu.sync_copy(cols_hbm.at[c, pl.ds(base + b * K, K)],
                            cols_v.at[g])

        def fire_gathers(src_tab, g):
            for j in range(K):
                pltpu.async_copy(src_tab.at[cols_v.at[g, j]],
                                 gbuf.at[g, j], gsem[g])

        def wait_gathers(src_tab, g):
            for j in range(K):
                pltpu.make_async_copy(src_tab.at[cols_v.at[g, j]],
                                      gbuf.at[g, j], gsem[g]).wait()

        def fire_scatters(g):
            for j in range(K):
                pltpu.async_copy(gbuf.at[g, j], acc.at[rows_v.at[g, j]],
                                 ssem[g], add=True)

        def wait_scatters(g):
            for j in range(K):
                pltpu.make_async_copy(gbuf.at[g, j], acc.at[rows_v.at[g, j]],
                                      ssem[g]).wait()

        load_idx(0, 0)
        fire_gathers(src_tab, 0)

        def body(bb, carry):
            for g in (0, 1):
                b = 2 * bb + g
                wait_gathers(src_tab, g)
                fire_scatters(g)
                o = 1 - g
                if g == 0:
                    @pl.when(bb >= 1)
                    def _():
                        wait_scatters(o)
                    load_idx(b + 1, o)
                    fire_gathers(src_tab, o)
                else:
                    @pl.when(bb < HBLOCKS - 1)
                    def _():
                        wait_scatters(o)
                        load_idx(b + 1, o)
                        fire_gathers(src_tab, o)
            return carry

        lax.fori_loop(0, HBLOCKS, body, 0)
        wait_scatters(0)
        wait_scatters(1)
        plsc.subcore_barrier()

        cps = []
        for k in range(NWB):
            r0 = s * ROWS_PT + k * WB
            cps.append(pltpu.async_copy(
                acc.at[pl.ds(r0, WB)], dst_tab.at[pl.ds(coff + r0, WB)],
                zsem))
        for cp in cps:
            cp.wait()
        plsc.subcore_barrier()

    layer(tab0, x1_out)
    layer(x1_out, x2_out)


def _mean_body(x0_ref, x1_ref, x2_ref, o_ref):
    o_ref[...] = (x0_ref[...] + x1_ref[...] + x2_ref[...]) * (1.0 / 3.0)


_MEAN_BLOCK = 3128


def _mean3(x0, x1, x2):
    n = x0.shape[0]
    grid = n // _MEAN_BLOCK
    spec = pl.BlockSpec((_MEAN_BLOCK, 128), lambda i: (i, 0))
    return pl.pallas_call(
        _mean_body,
        grid=(grid,),
        in_specs=[spec, spec, spec],
        out_specs=spec,
        out_shape=jax.ShapeDtypeStruct((n, 128), jnp.float32),
    )(x0, x1, x2)


def kernel(edge_index, emb_weight):
    row = edge_index[0]
    col = edge_index[1]
    pad = EPAD - E
    rows = jnp.concatenate(
        [row, jnp.full((pad,), NODES, jnp.int32)]).reshape(CHUNKS, CH)
    colp = jnp.concatenate([col, jnp.zeros((pad,), jnp.int32)])
    cols = jnp.stack([colp, colp + NP]).reshape(2, CHUNKS, CH)
    embp = jnp.pad(emb_weight, ((0, NP - NODES), (0, 0)))
    tab0 = jnp.concatenate([embp[:, :DH], embp[:, DH:]], axis=0)
    zeros = jnp.zeros((WB, DH), jnp.float32)

    x1_tab, x2_tab = _propagate(tab0, rows, cols, zeros)

    zt = _mean3(tab0.reshape(2 * NP * DH // 128, 128),
                x1_tab.reshape(2 * NP * DH // 128, 128),
                x2_tab.reshape(2 * NP * DH // 128, 128)).reshape(2 * NP, DH)
    z = jnp.concatenate([zt[:NODES], zt[NP:NP + NODES]], axis=1)
    return z[:NUSERS], z[NUSERS:NODES]


# speedup vs baseline: 1.0279x; 1.0279x over previous
import functools

import jax
import jax.numpy as jnp
from jax import lax
from jax.experimental import pallas as pl
from jax.experimental.pallas import tpu as pltpu
from jax.experimental.pallas import tpu_sc as plsc

NODES = 50000
NP = 50048
NUSERS = 25000
D = 64
DH = 32
E = 800000
CH = 128
CHUNKS = 6336
EPAD = CHUNKS * CH
TILES = 16
CPT = CHUNKS // TILES
K = 6
BLOCKS = CPT // K
ROWS_PT = NP // TILES
WB = 136
NWB = ROWS_PT // WB

_mesh = plsc.VectorSubcoreMesh(core_axis_name="c", subcore_axis_name="s")


@functools.partial(
    pl.kernel,
    mesh=_mesh,
    compiler_params=pltpu.CompilerParams(use_tc_tiling_on_sc=False),
    out_type=(
        jax.ShapeDtypeStruct((2 * NP, DH), jnp.float32),
        jax.ShapeDtypeStruct((2 * NP, DH), jnp.float32),
    ),
    scratch_types=[
        pltpu.VMEM((K, CH), jnp.int32),
        pltpu.VMEM((K, CH), jnp.int32),
        pltpu.VMEM((K, CH, DH), jnp.float32),
        pltpu.VMEM((WB, DH), jnp.float32),
        pltpu.VMEM_SHARED((NP, DH), jnp.float32),
        pltpu.SemaphoreType.DMA,
        pltpu.SemaphoreType.DMA,
    ],
)
def _propagate(tab0, rows_hbm, cols_hbm, zeros_hbm, x1_out, x2_out,
               rows_v, cols_v, gbuf, zbuf, acc, gsem, zsem):
    c = lax.axis_index("c")
    s = lax.axis_index("s")
    coff = c * NP

    pltpu.sync_copy(zeros_hbm, zbuf)

    def zero_acc():
        cps = []
        for k in range(NWB):
            cps.append(pltpu.async_copy(
                zbuf, acc.at[pl.ds(s * ROWS_PT + k * WB, WB)], zsem))
        for cp in cps:
            cp.wait()

    def layer(src_tab, dst_tab):
        zero_acc()
        plsc.subcore_barrier()

        base = s * CPT

        def body(b, carry):
            chunk0 = base + b * K
            pltpu.sync_copy(rows_hbm.at[pl.ds(chunk0, K)], rows_v)
            pltpu.sync_copy(cols_hbm.at[c, pl.ds(chunk0, K)], cols_v)
            cps = [
                pltpu.async_copy(src_tab.at[cols_v.at[j]], gbuf.at[j], gsem)
                for j in range(K)
            ]
            for cp in cps:
                cp.wait()
            for j in range(K):
                pltpu.sync_copy(gbuf.at[j], acc.at[rows_v.at[j]], add=True)
            return carry

        lax.fori_loop(0, BLOCKS, body, 0)
        plsc.subcore_barrier()

        cps = []
        for k in range(NWB):
            r0 = s * ROWS_PT + k * WB
            cps.append(pltpu.async_copy(
                acc.at[pl.ds(r0, WB)], dst_tab.at[pl.ds(coff + r0, WB)],
                zsem))
        for cp in cps:
            cp.wait()
        plsc.subcore_barrier()

    layer(tab0, x1_out)
    layer(x1_out, x2_out)


def _mean_body(x0_ref, x1_ref, x2_ref, o_ref):
    o_ref[...] = (x0_ref[...] + x1_ref[...] + x2_ref[...]) * (1.0 / 3.0)


_MEAN_BLOCK = 3128


def _mean3(x0, x1, x2):
    n = x0.shape[0]
    grid = n // _MEAN_BLOCK
    spec = pl.BlockSpec((_MEAN_BLOCK, 128), lambda i: (i, 0))
    return pl.pallas_call(
        _mean_body,
        grid=(grid,),
        in_specs=[spec, spec, spec],
        out_specs=spec,
        out_shape=jax.ShapeDtypeStruct((n, 128), jnp.float32),
    )(x0, x1, x2)


def kernel(edge_index, emb_weight):
    row = edge_index[0]
    col = edge_index[1]
    pad = EPAD - E
    rows = jnp.concatenate(
        [row, jnp.full((pad,), NODES, jnp.int32)]).reshape(CHUNKS, CH)
    colp = jnp.concatenate([col, jnp.zeros((pad,), jnp.int32)])
    cols = jnp.stack([colp, colp + NP]).reshape(2, CHUNKS, CH)
    embp = jnp.pad(emb_weight, ((0, NP - NODES), (0, 0)))
    tab0 = jnp.concatenate([embp[:, :DH], embp[:, DH:]], axis=0)
    zeros = jnp.zeros((WB, DH), jnp.float32)

    x1_tab, x2_tab = _propagate(tab0, rows, cols, zeros)

    zt = _mean3(tab0.reshape(2 * NP * DH // 128, 128),
                x1_tab.reshape(2 * NP * DH // 128, 128),
                x2_tab.reshape(2 * NP * DH // 128, 128)).reshape(2 * NP, DH)
    z = jnp.concatenate([zt[:NODES], zt[NP:NP + NODES]], axis=1)
    return z[:NUSERS], z[NUSERS:NODES]


# speedup vs baseline: 1.1377x; 1.1068x over previous
import functools

import jax
import jax.numpy as jnp
from jax import lax
from jax.experimental import pallas as pl
from jax.experimental.pallas import tpu as pltpu
from jax.experimental.pallas import tpu_sc as plsc

NODES = 50000
NP = 50048
NUSERS = 25000
D = 64
DH = 32
E = 800000
CH = 128
CHUNKS = 6272
EPAD = CHUNKS * CH
TILES = 16
CPT = CHUNKS // TILES
K = 4
BLOCKS = CPT // K
ROWS_PT = NP // TILES
WB = 136
NWB = ROWS_PT // WB

_mesh = plsc.VectorSubcoreMesh(core_axis_name="c", subcore_axis_name="s")


@functools.partial(
    pl.kernel,
    mesh=_mesh,
    compiler_params=pltpu.CompilerParams(use_tc_tiling_on_sc=False),
    out_type=(
        jax.ShapeDtypeStruct((2 * NP, DH), jnp.float32),
        jax.ShapeDtypeStruct((2 * NP, DH), jnp.float32),
    ),
    scratch_types=[
        pltpu.VMEM((K, CH), jnp.int32),
        pltpu.VMEM((K, CH), jnp.int32),
        pltpu.VMEM((K, CH, DH), jnp.float32),
        pltpu.VMEM((WB, DH), jnp.float32),
        pltpu.VMEM((WB, DH), jnp.float32),
        pltpu.VMEM_SHARED((NP, DH), jnp.float32),
        pltpu.SemaphoreType.DMA,
        pltpu.SemaphoreType.DMA,
    ],
)
def _propagate(tab0, rows_hbm, cols_hbm, zeros_hbm, x1_out, x2_out,
               rows_v, cols_v, gbuf, zbuf, wbuf, acc, gsem, zsem):
    c = lax.axis_index("c")
    s = lax.axis_index("s")
    coff = c * NP

    pltpu.sync_copy(zeros_hbm, zbuf)

    def zero_acc():
        cps = []
        for k in range(NWB):
            cps.append(pltpu.async_copy(
                zbuf, acc.at[pl.ds(s * ROWS_PT + k * WB, WB)], zsem))
        for cp in cps:
            cp.wait()

    def layer(src_tab, dst_tab):
        zero_acc()
        plsc.subcore_barrier()

        base = s * CPT

        def body(b, carry):
            chunk0 = base + b * K
            pltpu.sync_copy(rows_hbm.at[pl.ds(chunk0, K)], rows_v)
            pltpu.sync_copy(cols_hbm.at[c, pl.ds(chunk0, K)], cols_v)
            cps = [
                pltpu.async_copy(src_tab.at[cols_v.at[j]], gbuf.at[j], gsem)
                for j in range(K)
            ]
            for cp in cps:
                cp.wait()
            for j in range(K):
                pltpu.sync_copy(gbuf.at[j], acc.at[rows_v.at[j]], add=True)
            return carry

        lax.fori_loop(0, BLOCKS, body, 0)
        plsc.subcore_barrier()

        for k in range(NWB):
            r0 = s * ROWS_PT + k * WB
            pltpu.sync_copy(acc.at[pl.ds(r0, WB)], wbuf)
            pltpu.sync_copy(wbuf, dst_tab.at[pl.ds(coff + r0, WB)])
        plsc.subcore_barrier()

    layer(tab0, x1_out)
    layer(x1_out, x2_out)


def _mean_body(x0_ref, x1_ref, x2_ref, o_ref):
    o_ref[...] = (x0_ref[...] + x1_ref[...] + x2_ref[...]) * (1.0 / 3.0)


_MEAN_BLOCK = 3128


def _mean3(x0, x1, x2):
    n = x0.shape[0]
    grid = n // _MEAN_BLOCK
    spec = pl.BlockSpec((_MEAN_BLOCK, 128), lambda i: (i, 0))
    return pl.pallas_call(
        _mean_body,
        grid=(grid,),
        in_specs=[spec, spec, spec],
        out_specs=spec,
        out_shape=jax.ShapeDtypeStruct((n, 128), jnp.float32),
    )(x0, x1, x2)


def kernel(edge_index, emb_weight):
    row = edge_index[0]
    col = edge_index[1]
    pad = EPAD - E
    rows = jnp.concatenate(
        [row, jnp.full((pad,), NODES, jnp.int32)]).reshape(CHUNKS, CH)
    colp = jnp.concatenate([col, jnp.zeros((pad,), jnp.int32)])
    cols = jnp.stack([colp, colp + NP]).reshape(2, CHUNKS, CH)
    embp = jnp.pad(emb_weight, ((0, NP - NODES), (0, 0)))
    tab0 = jnp.concatenate([embp[:, :DH], embp[:, DH:]], axis=0)
    zeros = jnp.zeros((WB, DH), jnp.float32)

    x1_tab, x2_tab = _propagate(tab0, rows, cols, zeros)

    zt = _mean3(tab0.reshape(2 * NP * DH // 128, 128),
                x1_tab.reshape(2 * NP * DH // 128, 128),
                x2_tab.reshape(2 * NP * DH // 128, 128)).reshape(2 * NP, DH)
    z = jnp.concatenate([zt[:NODES], zt[NP:NP + NODES]], axis=1)
    return z[:NUSERS], z[NUSERS:NODES]
